# baseline (speedup 1.0000x reference)
import functools

import jax
import jax.numpy as jnp
from jax import lax
from jax.experimental import pallas as pl
from jax.experimental.pallas import tpu as pltpu
from jax.experimental.pallas import tpu_sc as plsc

B = 4096
T = 200
C = 64
VOCAB = 1000000
NC = 2
NS = 16
NW = NC * NS
BBLK = B // NW
NGBUF = 4
NOBUF = 4
LOOKAHEAD = 4
LANES = 16
JSTEP = BBLK // LANES
CA = C // 8


def _body(idx_hbm, tok_hbm, pos_hbm, out_hbm, idx_v, pos_v, *rest):
    gbuf = rest[:NGBUF]
    obuf = rest[NGBUF:NGBUF + NOBUF]
    gsem = rest[NGBUF + NOBUF:2 * NGBUF + NOBUF]
    ssem = rest[2 * NGBUF + NOBUF:2 * NGBUF + 2 * NOBUF]
    cid = lax.axis_index("c")
    sid = lax.axis_index("s")
    w = sid * NC + cid

    pltpu.sync_copy(idx_hbm.at[w], idx_v)
    pltpu.sync_copy(pos_hbm, pos_v)

    for n in range(LOOKAHEAD):
        pltpu.async_copy(tok_hbm.at[idx_v.at[n]], gbuf[n], gsem[n])

    lane = lax.iota(jnp.int32, LANES)

    def item(t, carry):
        for n in range(NGBUF):
            tt = NGBUF * t + n
            g = n
            o = n % NOBUF
            g2 = (n + LOOKAHEAD) % NGBUF

            pltpu.make_async_copy(
                tok_hbm.at[idx_v.at[tt]], gbuf[g], gsem[g]).wait()

            @pl.when(tt >= NOBUF)
            def _():
                for a in range(CA):
                    pltpu.make_async_copy(
                        obuf[o].at[a], out_hbm.at[0, a, pl.ds(0, 8)],
                        ssem[o]).wait()

            def rblock(rb, c2, _g=g, _o=o, _tt=tt):
                for rh in range(2):
                    r0 = (2 * rb + rh) * LANES
                    for cb in range(C // LANES):
                        cols = lane + cb * LANES
                        a_idx = lax.shift_right_logical(cols, 3)
                        ci_idx = lax.bitwise_and(cols, 7)
                        pvec = pos_v[_tt, pl.ds(cb * LANES, LANES)]
                        for k in range(LANES):
                            rows = r0 + lax.bitwise_and(lane + k, 15)
                            v = plsc.load_gather(gbuf[_g], [rows, cols])
                            plsc.store_scatter(
                                obuf[_o], [a_idx, ci_idx, rows], v + pvec)
                return c2

            lax.fori_loop(0, JSTEP // 2, rblock, 0)

            for a in range(CA):
                pltpu.async_copy(
                    obuf[o].at[a],
                    out_hbm.at[tt, a, pl.ds(w * 8, 8)],
                    ssem[o])

            @pl.when(tt < T - LOOKAHEAD)
            def _():
                pltpu.async_copy(
                    tok_hbm.at[idx_v.at[tt + LOOKAHEAD]], gbuf[g2], gsem[g2])
        return carry

    lax.fori_loop(0, T // NGBUF, item, 0)

    for o in range(NOBUF):
        for a in range(CA):
            pltpu.make_async_copy(
                obuf[o].at[a], out_hbm.at[0, a, pl.ds(0, 8)],
                ssem[o]).wait()


def _run(idx3, tok, pos):
    mesh = plsc.VectorSubcoreMesh(core_axis_name="c", subcore_axis_name="s")
    k = functools.partial(
        pl.kernel,
        mesh=mesh,
        out_type=jax.ShapeDtypeStruct((T, CA, NW * 8, BBLK), jnp.float32),
        scratch_types=(
            [pltpu.VMEM((T, BBLK), jnp.int32),
             pltpu.VMEM((T, C), jnp.float32)]
            + [pltpu.VMEM((BBLK, C), jnp.float32) for _ in range(NGBUF)]
            + [pltpu.VMEM((CA, 8, BBLK), jnp.float32) for _ in range(NOBUF)]
            + [pltpu.SemaphoreType.DMA for _ in range(NGBUF + NOBUF)]
        ),
        compiler_params=pltpu.CompilerParams(
            use_tc_tiling_on_sc=False, needs_layout_passes=False),
    )(_body)
    return k(idx3, tok, pos)


def kernel(idx, token_embedding_table, position_embedding_table):
    idx3 = jnp.transpose(
        idx.astype(jnp.int32).reshape(NW, BBLK, T), (0, 2, 1))
    t2 = lax.optimization_barrier(
        jnp.reshape(token_embedding_table, (VOCAB // 2, 2 * C)))
    tab = jnp.reshape(t2, (VOCAB, C))
    res = _run(idx3, tab, position_embedding_table)
    res5 = res.reshape(T, CA, NW, 8, BBLK)
    return jnp.transpose(res5, (2, 4, 0, 1, 3)).reshape(B, T, C)


# speedup vs baseline: 1.0436x; 1.0436x over previous
import functools

import jax
import jax.numpy as jnp
from jax import lax
from jax.experimental import pallas as pl
from jax.experimental.pallas import tpu as pltpu
from jax.experimental.pallas import tpu_sc as plsc

B = 4096
T = 200
C = 64
VOCAB = 1000000
NC = 2
NS = 16
NW = NC * NS
BBLK = B // NW
NGBUF = 4
NOBUF = 4
LOOKAHEAD = 4
LANES = 16
JSTEP = BBLK // LANES
CA = C // 8


def _body(idx_hbm, tok_hbm, pos_hbm, out_hbm, idx_v, pos_v, *rest):
    gbuf = rest[:NGBUF]
    obuf = rest[NGBUF:NGBUF + NOBUF]
    gsem = rest[NGBUF + NOBUF:2 * NGBUF + NOBUF]
    ssem = rest[2 * NGBUF + NOBUF:2 * NGBUF + 2 * NOBUF]
    cid = lax.axis_index("c")
    sid = lax.axis_index("s")
    w = sid * NC + cid

    pltpu.sync_copy(idx_hbm.at[w], idx_v)
    pltpu.sync_copy(pos_hbm, pos_v)

    for n in range(LOOKAHEAD):
        pltpu.async_copy(tok_hbm.at[idx_v.at[n]], gbuf[n], gsem[n])

    lane = lax.iota(jnp.int32, LANES)

    def item(t, carry):
        for n in range(NGBUF):
            tt = NGBUF * t + n
            g = n
            o = n % NOBUF
            g2 = (n + LOOKAHEAD) % NGBUF

            pltpu.make_async_copy(
                tok_hbm.at[idx_v.at[tt]], gbuf[g], gsem[g]).wait()

            @pl.when(tt >= NOBUF)
            def _():
                for a in range(CA):
                    pltpu.make_async_copy(
                        obuf[o].at[a], out_hbm.at[0, a, pl.ds(0, 8)],
                        ssem[o]).wait()

            def rblock(rb, c2, _g=g, _o=o, _tt=tt):
                r0 = rb * LANES
                for cb in range(C // LANES):
                    cols = lane + cb * LANES
                    a_idx = lax.shift_right_logical(cols, 3)
                    ci_idx = lax.bitwise_and(cols, 7)
                    pvec = pos_v[_tt, pl.ds(cb * LANES, LANES)]
                    for k in range(LANES):
                        rows = r0 + lax.bitwise_and(lane + k, 15)
                        v = plsc.load_gather(gbuf[_g], [rows, cols])
                        plsc.store_scatter(
                            obuf[_o], [a_idx, ci_idx, rows], v + pvec)
                return c2

            lax.fori_loop(0, JSTEP, rblock, 0)

            for a in range(CA):
                pltpu.async_copy(
                    obuf[o].at[a],
                    out_hbm.at[tt, a, pl.ds(w * 8, 8)],
                    ssem[o])

            @pl.when(tt < T - LOOKAHEAD)
            def _():
                pltpu.async_copy(
                    tok_hbm.at[idx_v.at[tt + LOOKAHEAD]], gbuf[g2], gsem[g2])
        return carry

    lax.fori_loop(0, T // NGBUF, item, 0)

    for o in range(NOBUF):
        for a in range(CA):
            pltpu.make_async_copy(
                obuf[o].at[a], out_hbm.at[0, a, pl.ds(0, 8)],
                ssem[o]).wait()


def _run(idx3, tok, pos):
    mesh = plsc.VectorSubcoreMesh(core_axis_name="c", subcore_axis_name="s")
    k = functools.partial(
        pl.kernel,
        mesh=mesh,
        out_type=jax.ShapeDtypeStruct((T, CA, NW * 8, BBLK), jnp.float32),
        scratch_types=(
            [pltpu.VMEM((T, BBLK), jnp.int32),
             pltpu.VMEM((T, C), jnp.float32)]
            + [pltpu.VMEM((BBLK, C), jnp.float32) for _ in range(NGBUF)]
            + [pltpu.VMEM((CA, 8, BBLK), jnp.float32) for _ in range(NOBUF)]
            + [pltpu.SemaphoreType.DMA for _ in range(NGBUF + NOBUF)]
        ),
        compiler_params=pltpu.CompilerParams(
            use_tc_tiling_on_sc=False, needs_layout_passes=False),
    )(_body)
    return k(idx3, tok, pos)


def kernel(idx, token_embedding_table, position_embedding_table):
    idx3 = jnp.transpose(
        idx.astype(jnp.int32).reshape(NW, BBLK, T), (0, 2, 1))
    res = _run(idx3, token_embedding_table, position_embedding_table)
    res5 = res.reshape(T, CA, NW, 8, BBLK)
    return jnp.transpose(res5, (2, 4, 0, 1, 3)).reshape(B, T, C)


# speedup vs baseline: 1.0513x; 1.0074x over previous
import functools

import jax
import jax.numpy as jnp
from jax import lax
from jax.experimental import pallas as pl
from jax.experimental.pallas import tpu as pltpu
from jax.experimental.pallas import tpu_sc as plsc

B = 4096
T = 200
C = 64
NC = 2
NS = 16
NW = NC * NS
BBLK = B // NW
NGBUF = 4
NOBUF = 4
LOOKAHEAD = 4
LANES = 16
JSTEP = BBLK // LANES
CA = C // 8


def _body(idx_hbm, tok_hbm, pos_hbm, out_hbm, idx_v, pos_v, *rest):
    gbuf = rest[:NGBUF]
    obuf = rest[NGBUF:NGBUF + NOBUF]
    gsem = rest[NGBUF + NOBUF:2 * NGBUF + NOBUF]
    ssem = rest[2 * NGBUF + NOBUF:2 * NGBUF + 2 * NOBUF]
    cid = lax.axis_index("c")
    sid = lax.axis_index("s")
    w = sid * NC + cid

    pltpu.sync_copy(idx_hbm.at[w], idx_v)
    pltpu.sync_copy(pos_hbm, pos_v)

    for n in range(LOOKAHEAD):
        pltpu.async_copy(tok_hbm.at[idx_v.at[n]], gbuf[n], gsem[n])

    lane = lax.iota(jnp.int32, LANES)

    def item(t, carry):
        for n in range(NGBUF):
            tt = NGBUF * t + n
            g = n
            o = n % NOBUF
            g2 = (n + LOOKAHEAD) % NGBUF

            pltpu.make_async_copy(
                tok_hbm.at[idx_v.at[tt]], gbuf[g], gsem[g]).wait()

            @pl.when(tt >= NOBUF)
            def _():
                for a in range(CA):
                    pltpu.make_async_copy(
                        obuf[o].at[a], out_hbm.at[0, a, pl.ds(0, 8)],
                        ssem[o]).wait()

            def rblock(rb, c2, _g=g, _o=o, _tt=tt):
                r0 = rb * LANES
                rows_k = [r0 + lax.bitwise_and(lane + k, 15)
                          for k in range(LANES)]
                for cb in range(C // LANES):
                    cols = lane + cb * LANES
                    a_idx = lax.shift_right_logical(cols, 3)
                    ci_idx = lax.bitwise_and(cols, 7)
                    pvec = pos_v[_tt, pl.ds(cb * LANES, LANES)]
                    for k in range(LANES):
                        rows = rows_k[k]
                        v = plsc.load_gather(gbuf[_g], [rows, cols])
                        plsc.store_scatter(
                            obuf[_o], [a_idx, ci_idx, rows], v + pvec)
                return c2

            lax.fori_loop(0, JSTEP, rblock, 0)

            for a in range(CA):
                pltpu.async_copy(
                    obuf[o].at[a],
                    out_hbm.at[tt, a, pl.ds(w * 8, 8)],
                    ssem[o])

            @pl.when(tt < T - LOOKAHEAD)
            def _():
                pltpu.async_copy(
                    tok_hbm.at[idx_v.at[tt + LOOKAHEAD]], gbuf[g2], gsem[g2])
        return carry

    lax.fori_loop(0, T // NGBUF, item, 0)

    for o in range(NOBUF):
        for a in range(CA):
            pltpu.make_async_copy(
                obuf[o].at[a], out_hbm.at[0, a, pl.ds(0, 8)],
                ssem[o]).wait()


def _run(idx3, tok, pos):
    mesh = plsc.VectorSubcoreMesh(core_axis_name="c", subcore_axis_name="s")
    k = functools.partial(
        pl.kernel,
        mesh=mesh,
        out_type=jax.ShapeDtypeStruct((T, CA, NW * 8, BBLK), jnp.float32),
        scratch_types=(
            [pltpu.VMEM((T, BBLK), jnp.int32),
             pltpu.VMEM((T, C), jnp.float32)]
            + [pltpu.VMEM((BBLK, C), jnp.float32) for _ in range(NGBUF)]
            + [pltpu.VMEM((CA, 8, BBLK), jnp.float32) for _ in range(NOBUF)]
            + [pltpu.SemaphoreType.DMA for _ in range(NGBUF + NOBUF)]
        ),
        compiler_params=pltpu.CompilerParams(
            use_tc_tiling_on_sc=False, needs_layout_passes=False),
    )(_body)
    return k(idx3, tok, pos)


def kernel(idx, token_embedding_table, position_embedding_table):
    idx3 = jnp.transpose(
        idx.astype(jnp.int32).reshape(NW, BBLK, T), (0, 2, 1))
    res = _run(idx3, token_embedding_table, position_embedding_table)
    res5 = res.reshape(T, CA, NW, 8, BBLK)
    return jnp.transpose(res5, (2, 4, 0, 1, 3)).reshape(B, T, C)
